# baseline (speedup 1.0000x reference)
import jax
import jax.numpy as jnp
import numpy as np
from jax.experimental import pallas as pl
from jax.experimental.pallas import tpu as pltpu

_REPARAM_OFFSET = 2.0 ** -18
_PEDESTAL = _REPARAM_OFFSET ** 2
_BETA_MIN = 1e-6
_BOUND_BETA = float((_BETA_MIN + _PEDESTAL) ** 0.5)
_BOUND_A = float(_PEDESTAL ** 0.5)

_HB = 32


def _gdn_body(x_ref, a1_ref, a2_ref, o_ref, t_ref):
    c8, w = a2_ref.shape[0], x_ref.shape[3]
    r8 = a1_ref.shape[0]
    x = x_ref[0].reshape(c8, w)
    x2 = x * x
    t_ref[0:r8, :] = jnp.dot(a1_ref[...], x2, preferred_element_type=jnp.float32)
    ones_row = jax.lax.broadcasted_iota(jnp.int32, (8, w), 0) == 0
    t_ref[r8:r8 + 8, :] = jnp.where(ones_row, 1.0, 0.0)
    denom = jnp.dot(a2_ref[...], t_ref[...], preferred_element_type=jnp.float32)
    out = x * jax.lax.rsqrt(denom)
    o_ref[0] = out.reshape(c8 // _HB, _HB, w)


def kernel(x, beta, A):
    N, C, H, W = x.shape
    R = A.shape[1]
    r8, c8 = R * _HB, C * _HB

    beta_r = jnp.maximum(beta, _BOUND_BETA) ** 2 - _PEDESTAL
    a_r = jnp.maximum(A, _BOUND_A) ** 2 - _PEDESTAL
    eye_hb = jnp.eye(_HB, dtype=jnp.float32)
    big_a1 = jnp.kron(a_r.T, eye_hb)
    big_a2 = jnp.kron(a_r, eye_hb)
    beta_col = jnp.repeat(beta_r, _HB)[:, None]
    pad = jnp.zeros((c8, 7), jnp.float32)
    big_a2 = jnp.concatenate([big_a2, beta_col, pad], axis=1)

    return pl.pallas_call(
        _gdn_body,
        grid=(N, H // _HB),
        in_specs=[
            pl.BlockSpec((1, C, _HB, W), lambda n, h: (n, 0, h, 0)),
            pl.BlockSpec((r8, c8), lambda n, h: (0, 0)),
            pl.BlockSpec((c8, r8 + 8), lambda n, h: (0, 0)),
        ],
        out_specs=pl.BlockSpec((1, C, _HB, W), lambda n, h: (n, 0, h, 0)),
        out_shape=jax.ShapeDtypeStruct((N, C, H, W), x.dtype),
        scratch_shapes=[pltpu.VMEM((r8 + 8, W), jnp.float32)],
        compiler_params=pltpu.CompilerParams(
            dimension_semantics=("parallel", "parallel"),
            vmem_limit_bytes=50 * 1024 * 1024,
        ),
    )(x, big_a1, big_a2)


# speedup vs baseline: 1.3064x; 1.3064x over previous
import jax
import jax.numpy as jnp
import numpy as np
from jax.experimental import pallas as pl
from jax.experimental.pallas import tpu as pltpu

_REPARAM_OFFSET = 2.0 ** -18
_PEDESTAL = _REPARAM_OFFSET ** 2
_BETA_MIN = 1e-6
_BOUND_BETA = float((_BETA_MIN + _PEDESTAL) ** 0.5)
_BOUND_A = float(_PEDESTAL ** 0.5)

_HB = 32
_HS = 16


def _gdn_body(x_ref, a1_ref, a2_ref, o_ref, t_ref):
    cs, w = a2_ref.shape[0], x_ref.shape[3]
    rs = a1_ref.shape[0]
    ones_row = jax.lax.broadcasted_iota(jnp.int32, (8, w), 0) == 0
    t_ref[rs:rs + 8, :] = jnp.where(ones_row, 1.0, 0.0)
    for k in range(_HB // _HS):
        x = x_ref[0][:, k * _HS:(k + 1) * _HS, :].reshape(cs, w)
        x2 = x * x
        t_ref[0:rs, :] = jnp.dot(a1_ref[...], x2, preferred_element_type=jnp.float32)
        denom = jnp.dot(a2_ref[...], t_ref[...], preferred_element_type=jnp.float32)
        out = x * jax.lax.rsqrt(denom)
        o_ref[0, :, k * _HS:(k + 1) * _HS, :] = out.reshape(cs // _HS, _HS, w)


def kernel(x, beta, A):
    N, C, H, W = x.shape
    R = A.shape[1]
    r8, c8 = R * _HS, C * _HS

    beta_r = jnp.maximum(beta, _BOUND_BETA) ** 2 - _PEDESTAL
    a_r = jnp.maximum(A, _BOUND_A) ** 2 - _PEDESTAL
    eye_hs = jnp.eye(_HS, dtype=jnp.float32)
    big_a1 = jnp.kron(a_r.T, eye_hs)
    big_a2 = jnp.kron(a_r, eye_hs)
    beta_col = jnp.repeat(beta_r, _HS)[:, None]
    pad = jnp.zeros((c8, 7), jnp.float32)
    big_a2 = jnp.concatenate([big_a2, beta_col, pad], axis=1)

    return pl.pallas_call(
        _gdn_body,
        grid=(N, H // _HB),
        in_specs=[
            pl.BlockSpec((1, C, _HB, W), lambda n, h: (n, 0, h, 0)),
            pl.BlockSpec((r8, c8), lambda n, h: (0, 0)),
            pl.BlockSpec((c8, r8 + 8), lambda n, h: (0, 0)),
        ],
        out_specs=pl.BlockSpec((1, C, _HB, W), lambda n, h: (n, 0, h, 0)),
        out_shape=jax.ShapeDtypeStruct((N, C, H, W), x.dtype),
        scratch_shapes=[pltpu.VMEM((r8 + 8, W), jnp.float32)],
        compiler_params=pltpu.CompilerParams(
            dimension_semantics=("parallel", "parallel"),
            vmem_limit_bytes=50 * 1024 * 1024,
        ),
    )(x, big_a1, big_a2)


# speedup vs baseline: 1.3249x; 1.0142x over previous
import jax
import jax.numpy as jnp
import numpy as np
from jax.experimental import pallas as pl
from jax.experimental.pallas import tpu as pltpu

_REPARAM_OFFSET = 2.0 ** -18
_PEDESTAL = _REPARAM_OFFSET ** 2
_BETA_MIN = 1e-6
_BOUND_BETA = float((_BETA_MIN + _PEDESTAL) ** 0.5)
_BOUND_A = float(_PEDESTAL ** 0.5)

_HB = 64
_HS = 16


def _gdn_body(x_ref, a1_ref, a2_ref, o_ref, t_ref):
    cs, w = a2_ref.shape[0], x_ref.shape[3]
    rs = a1_ref.shape[0]
    ones_row = jax.lax.broadcasted_iota(jnp.int32, (8, w), 0) == 0
    t_ref[rs:rs + 8, :] = jnp.where(ones_row, 1.0, 0.0)
    for k in range(_HB // _HS):
        x = x_ref[0][:, k * _HS:(k + 1) * _HS, :].reshape(cs, w)
        x2 = x * x
        t_ref[0:rs, :] = jnp.dot(a1_ref[...], x2, preferred_element_type=jnp.float32)
        denom = jnp.dot(a2_ref[...], t_ref[...], preferred_element_type=jnp.float32)
        out = x * jax.lax.rsqrt(denom)
        o_ref[0, :, k * _HS:(k + 1) * _HS, :] = out.reshape(cs // _HS, _HS, w)


def kernel(x, beta, A):
    N, C, H, W = x.shape
    R = A.shape[1]
    r8, c8 = R * _HS, C * _HS

    beta_r = jnp.maximum(beta, _BOUND_BETA) ** 2 - _PEDESTAL
    a_r = jnp.maximum(A, _BOUND_A) ** 2 - _PEDESTAL
    eye_hs = jnp.eye(_HS, dtype=jnp.float32)
    big_a1 = jnp.kron(a_r.T, eye_hs)
    big_a2 = jnp.kron(a_r, eye_hs)
    beta_col = jnp.repeat(beta_r, _HS)[:, None]
    pad = jnp.zeros((c8, 7), jnp.float32)
    big_a2 = jnp.concatenate([big_a2, beta_col, pad], axis=1)

    return pl.pallas_call(
        _gdn_body,
        grid=(N, H // _HB),
        in_specs=[
            pl.BlockSpec((1, C, _HB, W), lambda n, h: (n, 0, h, 0)),
            pl.BlockSpec((r8, c8), lambda n, h: (0, 0)),
            pl.BlockSpec((c8, r8 + 8), lambda n, h: (0, 0)),
        ],
        out_specs=pl.BlockSpec((1, C, _HB, W), lambda n, h: (n, 0, h, 0)),
        out_shape=jax.ShapeDtypeStruct((N, C, H, W), x.dtype),
        scratch_shapes=[pltpu.VMEM((r8 + 8, W), jnp.float32)],
        compiler_params=pltpu.CompilerParams(
            dimension_semantics=("parallel", "parallel"),
            vmem_limit_bytes=62 * 1024 * 1024,
        ),
    )(x, big_a1, big_a2)


# speedup vs baseline: 1.4021x; 1.0583x over previous
import jax
import jax.numpy as jnp
from jax.experimental import pallas as pl
from jax.experimental.pallas import tpu as pltpu

_REPARAM_OFFSET = 2.0 ** -18
_PEDESTAL = _REPARAM_OFFSET ** 2
_BETA_MIN = 1e-6
_BOUND_BETA = float((_BETA_MIN + _PEDESTAL) ** 0.5)
_BOUND_A = float(_PEDESTAL ** 0.5)

_HB = 64
_HS = 8


def _gdn_body(x_ref, a1_ref, a2_ref, o_ref, t_ref):
    cs, w = a2_ref.shape[0], x_ref.shape[3]
    rs = a1_ref.shape[0]
    ones_row = jax.lax.broadcasted_iota(jnp.int32, (8, w), 0) == 0
    t_ref[rs:rs + 8, :] = jnp.where(ones_row, 1.0, 0.0)
    for k in range(_HB // _HS):
        x = x_ref[0][:, k * _HS:(k + 1) * _HS, :].reshape(cs, w)
        x2 = x * x
        t_ref[0:rs, :] = jnp.dot(a1_ref[...], x2, preferred_element_type=jnp.float32)
        denom = jnp.dot(a2_ref[...], t_ref[...], preferred_element_type=jnp.float32)
        out = x * jax.lax.rsqrt(denom)
        o_ref[0, :, k * _HS:(k + 1) * _HS, :] = out.reshape(cs // _HS, _HS, w)


def kernel(x, beta, A):
    N, C, H, W = x.shape
    R = A.shape[1]
    r8, c8 = R * _HS, C * _HS

    beta_r = jnp.maximum(beta, _BOUND_BETA) ** 2 - _PEDESTAL
    a_r = jnp.maximum(A, _BOUND_A) ** 2 - _PEDESTAL
    eye_hs = jnp.eye(_HS, dtype=jnp.float32)
    big_a1 = jnp.kron(a_r.T, eye_hs)
    big_a2 = jnp.kron(a_r, eye_hs)
    beta_col = jnp.repeat(beta_r, _HS)[:, None]
    pad = jnp.zeros((c8, 7), jnp.float32)
    big_a2 = jnp.concatenate([big_a2, beta_col, pad], axis=1)

    return pl.pallas_call(
        _gdn_body,
        grid=(N, H // _HB),
        in_specs=[
            pl.BlockSpec((1, C, _HB, W), lambda n, h: (n, 0, h, 0)),
            pl.BlockSpec((r8, c8), lambda n, h: (0, 0)),
            pl.BlockSpec((c8, r8 + 8), lambda n, h: (0, 0)),
        ],
        out_specs=pl.BlockSpec((1, C, _HB, W), lambda n, h: (n, 0, h, 0)),
        out_shape=jax.ShapeDtypeStruct((N, C, H, W), x.dtype),
        scratch_shapes=[pltpu.VMEM((r8 + 8, W), jnp.float32)],
        compiler_params=pltpu.CompilerParams(
            dimension_semantics=("parallel", "parallel"),
            vmem_limit_bytes=62 * 1024 * 1024,
        ),
    )(x, big_a1, big_a2)


# speedup vs baseline: 1.4023x; 1.0001x over previous
import jax
import jax.numpy as jnp
from jax.experimental import pallas as pl
from jax.experimental.pallas import tpu as pltpu

_REPARAM_OFFSET = 2.0 ** -18
_PEDESTAL = _REPARAM_OFFSET ** 2
_BETA_MIN = 1e-6
_BOUND_BETA = float((_BETA_MIN + _PEDESTAL) ** 0.5)
_BOUND_A = float(_PEDESTAL ** 0.5)

_HB = 64
_HS = 8


def _gdn_body(x_ref, a1_ref, a2_ref, o_ref, t_ref):
    cs, w = a2_ref.shape[0], x_ref.shape[3]
    rs = a1_ref.shape[0]
    ones_row = jax.lax.broadcasted_iota(jnp.int32, (8, w), 0) == 0
    ones_val = jnp.where(ones_row, 1.0, 0.0)
    t_ref[0, rs:rs + 8, :] = ones_val
    t_ref[1, rs:rs + 8, :] = ones_val
    for k in range(_HB // _HS):
        x = x_ref[0][:, k * _HS:(k + 1) * _HS, :].reshape(cs, w)
        x2 = x * x
        t_ref[k % 2, 0:rs, :] = jnp.dot(a1_ref[...], x2, preferred_element_type=jnp.float32)
        denom = jnp.dot(a2_ref[...], t_ref[k % 2], preferred_element_type=jnp.float32)
        out = x * jax.lax.rsqrt(denom)
        o_ref[0, :, k * _HS:(k + 1) * _HS, :] = out.reshape(cs // _HS, _HS, w)


def kernel(x, beta, A):
    N, C, H, W = x.shape
    R = A.shape[1]
    r8, c8 = R * _HS, C * _HS

    beta_r = jnp.maximum(beta, _BOUND_BETA) ** 2 - _PEDESTAL
    a_r = jnp.maximum(A, _BOUND_A) ** 2 - _PEDESTAL
    eye_hs = jnp.eye(_HS, dtype=jnp.float32)
    big_a1 = jnp.kron(a_r.T, eye_hs)
    big_a2 = jnp.kron(a_r, eye_hs)
    beta_col = jnp.repeat(beta_r, _HS)[:, None]
    pad = jnp.zeros((c8, 7), jnp.float32)
    big_a2 = jnp.concatenate([big_a2, beta_col, pad], axis=1)

    return pl.pallas_call(
        _gdn_body,
        grid=(N, H // _HB),
        in_specs=[
            pl.BlockSpec((1, C, _HB, W), lambda n, h: (n, 0, h, 0)),
            pl.BlockSpec((r8, c8), lambda n, h: (0, 0)),
            pl.BlockSpec((c8, r8 + 8), lambda n, h: (0, 0)),
        ],
        out_specs=pl.BlockSpec((1, C, _HB, W), lambda n, h: (n, 0, h, 0)),
        out_shape=jax.ShapeDtypeStruct((N, C, H, W), x.dtype),
        scratch_shapes=[pltpu.VMEM((2, r8 + 8, W), jnp.float32)],
        compiler_params=pltpu.CompilerParams(
            dimension_semantics=("parallel", "parallel"),
            vmem_limit_bytes=62 * 1024 * 1024,
        ),
    )(x, big_a1, big_a2)


# speedup vs baseline: 1.4327x; 1.0216x over previous
import jax
import jax.numpy as jnp
from jax.experimental import pallas as pl
from jax.experimental.pallas import tpu as pltpu

_REPARAM_OFFSET = 2.0 ** -18
_PEDESTAL = _REPARAM_OFFSET ** 2
_BETA_MIN = 1e-6
_BOUND_BETA = float((_BETA_MIN + _PEDESTAL) ** 0.5)
_BOUND_A = float(_PEDESTAL ** 0.5)

_HB = 64
_HS = 8


def _gdn_body(x_ref, a1_ref, a2_ref, o_ref, t_ref):
    cs, w = a2_ref.shape[0], x_ref.shape[3]
    rs = a1_ref.shape[0]
    ones_row = jax.lax.broadcasted_iota(jnp.int32, (8, w), 0) == 0
    t_ref[rs:rs + 8, :] = jnp.where(ones_row, 1.0, 0.0).astype(jnp.bfloat16)
    for k in range(_HB // _HS):
        x = x_ref[0][:, k * _HS:(k + 1) * _HS, :].reshape(cs, w)
        x2 = (x * x).astype(jnp.bfloat16)
        t = jnp.dot(a1_ref[...], x2, preferred_element_type=jnp.float32)
        t_ref[0:rs, :] = t.astype(jnp.bfloat16)
        denom = jnp.dot(a2_ref[...], t_ref[...], preferred_element_type=jnp.float32)
        out = x * jax.lax.rsqrt(denom)
        o_ref[0, :, k * _HS:(k + 1) * _HS, :] = out.reshape(cs // _HS, _HS, w)


def kernel(x, beta, A):
    N, C, H, W = x.shape
    R = A.shape[1]
    r8, c8 = R * _HS, C * _HS

    beta_r = jnp.maximum(beta, _BOUND_BETA) ** 2 - _PEDESTAL
    a_r = jnp.maximum(A, _BOUND_A) ** 2 - _PEDESTAL
    eye_hs = jnp.eye(_HS, dtype=jnp.float32)
    big_a1 = jnp.kron(a_r.T, eye_hs)
    big_a2 = jnp.kron(a_r, eye_hs)
    beta_col = jnp.repeat(beta_r, _HS)[:, None]
    pad = jnp.zeros((c8, 7), jnp.float32)
    big_a2 = jnp.concatenate([big_a2, beta_col, pad], axis=1)

    return pl.pallas_call(
        _gdn_body,
        grid=(N, H // _HB),
        in_specs=[
            pl.BlockSpec((1, C, _HB, W), lambda n, h: (n, 0, h, 0)),
            pl.BlockSpec((r8, c8), lambda n, h: (0, 0)),
            pl.BlockSpec((c8, r8 + 8), lambda n, h: (0, 0)),
        ],
        out_specs=pl.BlockSpec((1, C, _HB, W), lambda n, h: (n, 0, h, 0)),
        out_shape=jax.ShapeDtypeStruct((N, C, H, W), x.dtype),
        scratch_shapes=[pltpu.VMEM((r8 + 8, W), jnp.bfloat16)],
        compiler_params=pltpu.CompilerParams(
            dimension_semantics=("parallel", "parallel"),
            vmem_limit_bytes=62 * 1024 * 1024,
        ),
    )(x, big_a1.astype(jnp.bfloat16), big_a2.astype(jnp.bfloat16))


# speedup vs baseline: 1.4352x; 1.0018x over previous
import jax
import jax.numpy as jnp
from jax.experimental import pallas as pl
from jax.experimental.pallas import tpu as pltpu

_REPARAM_OFFSET = 2.0 ** -18
_PEDESTAL = _REPARAM_OFFSET ** 2
_BETA_MIN = 1e-6
_BOUND_BETA = float((_BETA_MIN + _PEDESTAL) ** 0.5)
_BOUND_A = float(_PEDESTAL ** 0.5)

_HB = 64
_HS = 8


def _gdn_body(x_ref, a1_ref, a2_ref, o_ref):
    cs, w = a2_ref.shape[0], x_ref.shape[3]
    rs = a1_ref.shape[0]
    ones_row = jax.lax.broadcasted_iota(jnp.int32, (8, w), 0) == 0
    ones_bf = jnp.where(ones_row, 1.0, 0.0).astype(jnp.bfloat16)
    for k in range(_HB // _HS):
        x = x_ref[0][:, k * _HS:(k + 1) * _HS, :].reshape(cs, w)
        x2 = (x * x).astype(jnp.bfloat16)
        t = jnp.dot(a1_ref[...], x2, preferred_element_type=jnp.float32)
        t_aug = jnp.concatenate([t.astype(jnp.bfloat16), ones_bf], axis=0)
        denom = jnp.dot(a2_ref[...], t_aug, preferred_element_type=jnp.float32)
        out = x * jax.lax.rsqrt(denom)
        o_ref[0, :, k * _HS:(k + 1) * _HS, :] = out.reshape(cs // _HS, _HS, w)


def kernel(x, beta, A):
    N, C, H, W = x.shape
    R = A.shape[1]
    r8, c8 = R * _HS, C * _HS

    beta_r = jnp.maximum(beta, _BOUND_BETA) ** 2 - _PEDESTAL
    a_r = jnp.maximum(A, _BOUND_A) ** 2 - _PEDESTAL
    eye_hs = jnp.eye(_HS, dtype=jnp.float32)
    big_a1 = jnp.kron(a_r.T, eye_hs)
    big_a2 = jnp.kron(a_r, eye_hs)
    beta_col = jnp.repeat(beta_r, _HS)[:, None]
    pad = jnp.zeros((c8, 7), jnp.float32)
    big_a2 = jnp.concatenate([big_a2, beta_col, pad], axis=1)

    return pl.pallas_call(
        _gdn_body,
        grid=(N, H // _HB),
        in_specs=[
            pl.BlockSpec((1, C, _HB, W), lambda n, h: (n, 0, h, 0)),
            pl.BlockSpec((r8, c8), lambda n, h: (0, 0)),
            pl.BlockSpec((c8, r8 + 8), lambda n, h: (0, 0)),
        ],
        out_specs=pl.BlockSpec((1, C, _HB, W), lambda n, h: (n, 0, h, 0)),
        out_shape=jax.ShapeDtypeStruct((N, C, H, W), x.dtype),
        compiler_params=pltpu.CompilerParams(
            dimension_semantics=("parallel", "parallel"),
            vmem_limit_bytes=62 * 1024 * 1024,
        ),
    )(x, big_a1.astype(jnp.bfloat16), big_a2.astype(jnp.bfloat16))
